# baseline (speedup 1.0000x reference)
import functools

import jax
import jax.numpy as jnp
from jax import lax
from jax.experimental import pallas as pl
from jax.experimental.pallas import tpu as pltpu
from jax.experimental.pallas import tpu_sc as plsc

_B, _M, _C = 2, 4096, 16
_W = _C + _M
_R = 256
_NT = _M // _R
_G = 512
_NS = 16
_MS = _M // _NS


def _self_flags(cid2, lt2, clen2):
    mesh = plsc.VectorSubcoreMesh(core_axis_name="c", subcore_axis_name="s")

    @functools.partial(
        pl.kernel,
        mesh=mesh,
        out_type=jax.ShapeDtypeStruct((_B, 1, _M), jnp.float32),
        compiler_params=pltpu.CompilerParams(needs_layout_passes=False),
        scratch_types=[
            pltpu.VMEM((_MS,), jnp.int32),
            pltpu.VMEM((_G,), jnp.float32),
            pltpu.VMEM((_NS, _G), jnp.float32),
            pltpu.VMEM((_MS * _C,), jnp.int32),
            pltpu.VMEM((_MS,), jnp.int32),
            pltpu.VMEM((_MS,), jnp.float32),
            pltpu.VMEM_SHARED((_NS, _G), jnp.float32),
        ],
    )
    def k(cid_hbm, lt_hbm, clen_hbm, out_hbm,
          cids_v, hist_v, histall_v, lt_v, clen_v, self_v, hist_sh):
        b = lax.axis_index("c")
        sid = lax.axis_index("s")
        base = sid * _MS

        pltpu.sync_copy(cid_hbm.at[b, pl.ds(base, _MS)], cids_v)
        pltpu.sync_copy(lt_hbm.at[b, pl.ds(base * _C, _MS * _C)], lt_v)
        pltpu.sync_copy(clen_hbm.at[b, pl.ds(base, _MS)], clen_v)

        ones = jnp.ones((16,), jnp.float32)
        zeros = jnp.zeros((16,), jnp.float32)
        for g in range(_G // 16):
            hist_v[pl.ds(g * 16, 16)] = zeros
        for kk in range(_MS // 16):
            ids = cids_v[pl.ds(kk * 16, 16)]
            plsc.addupdate_scatter(hist_v, [ids], ones)

        pltpu.sync_copy(hist_v, hist_sh.at[sid])
        plsc.subcore_barrier()
        pltpu.sync_copy(hist_sh, histall_v)
        for g in range(_G // 16):
            acc = histall_v[0, pl.ds(g * 16, 16)]
            for r in range(1, _NS):
                acc = acc + histall_v[r, pl.ds(g * 16, 16)]
            hist_v[pl.ds(g * 16, 16)] = acc

        lane = lax.broadcasted_iota(jnp.int32, (16,), 0)
        for kk in range(_MS // 16):
            ids = cids_v[pl.ds(kk * 16, 16)]
            sizes = plsc.load_gather(hist_v, [ids])
            clen_g = clen_v[pl.ds(kk * 16, 16)]
            mi = lane + (kk * 16)
            lcnt = jnp.zeros((16,), jnp.float32)
            for c in range(_C):
                vals = plsc.load_gather(lt_v, [mi * _C + c])
                hitc = jnp.logical_and(vals != 0, clen_g > c)
                lcnt = lcnt + jnp.where(hitc, 1.0, 0.0)
            nf = (sizes - 1.0) + lcnt
            self_v[pl.ds(kk * 16, 16)] = jnp.where(nf == 0.0, 1.0, 0.0)

        pltpu.sync_copy(self_v, out_hbm.at[b, 0, pl.ds(base, _MS)])

    return k(cid2, lt2, clen2)


def _loss_body(scores_ref, lt_ref, clen_ref, cid_tile_ref,
               cid_col_ref, out_ref, base_ref, ediag_ref, sdiag_ref):
    b = pl.program_id(0)
    t = pl.program_id(1)

    rr = jax.lax.broadcasted_iota(jnp.int32, (_R, _R), 0)
    cc = jax.lax.broadcasted_iota(jnp.int32, (_R, _R), 1)
    eye_rr = rr == cc
    for k in range(_NT):
        @pl.when(t == k)
        def _extract(k=k):
            win = scores_ref[0, (_C + k * _R):(_C + (k + 1) * _R), :]
            sdiag_ref[...] = jnp.sum(jnp.where(eye_rr, win, 0.0), axis=0,
                                     keepdims=True)

    s = scores_ref[0]
    lt = lt_ref[0]
    clen = clen_ref[0]
    cid_row = cid_tile_ref[0]
    cid_col = cid_col_ref[0]

    m = jnp.max(s, axis=0, keepdims=True)
    e = jnp.exp(s - m)
    sum_e = jnp.sum(e, axis=0, keepdims=True)

    e_c = e[_C:, :]
    same = cid_col == cid_row
    sum_same_e = jnp.sum(jnp.where(same, e_c, 0.0), axis=0, keepdims=True)

    e_diag = jnp.exp(sdiag_ref[...] - m)
    sum_mates_e = jnp.maximum(sum_same_e - e_diag, 0.0)

    c16 = jax.lax.broadcasted_iota(jnp.int32, (_C, _R), 0)
    e_l = e[:_C, :]
    link_valid = c16 < clen
    sum_inv_l = jnp.sum(jnp.where(link_valid, 0.0, e_l), axis=0, keepdims=True)
    gold_l = jnp.logical_and(lt != 0, link_valid)
    sum_gold_l = jnp.sum(jnp.where(gold_l, e_l, 0.0), axis=0, keepdims=True)

    sum_all = sum_e - sum_inv_l

    base_ref[...] = (sum_mates_e + sum_gold_l).reshape(1, 1, _R)
    ediag_ref[...] = e_diag.reshape(1, 1, _R)

    contrib = jnp.sum(jnp.log(sum_all), axis=1, keepdims=True)

    @pl.when(jnp.logical_and(b == 0, t == 0))
    def _init():
        out_ref[...] = jnp.zeros((1, 1), jnp.float32)

    out_ref[...] += contrib


def _combine_body(base_ref, ediag_ref, self_ref, out_ref):
    b = pl.program_id(0)
    sum_gold = base_ref[0] + self_ref[0] * ediag_ref[0]
    contrib = jnp.sum(jnp.log(sum_gold), axis=1, keepdims=True)

    @pl.when(b == 0)
    def _init():
        out_ref[...] = jnp.zeros((1, 1), jnp.float32)

    out_ref[...] += contrib


@jax.jit
def kernel(scores, linker_targets, candidate_lengths, cluster_ids):
    B, M, W = scores.shape
    C = W - M
    scores_t = jnp.transpose(scores, (0, 2, 1))
    lt_t = jnp.transpose(linker_targets, (0, 2, 1))
    clen2 = candidate_lengths.reshape(B, 1, M)
    cid2 = cluster_ids.reshape(B, 1, M)
    cid_col = cluster_ids.reshape(B, M, 1)

    self_f = _self_flags(
        cluster_ids,
        linker_targets.reshape(B, M * C),
        candidate_lengths,
    )

    grid = (B, M // _R)
    out1, base, ediag = pl.pallas_call(
        _loss_body,
        grid=grid,
        in_specs=[
            pl.BlockSpec((1, W, _R), lambda b, t: (b, 0, t)),
            pl.BlockSpec((1, C, _R), lambda b, t: (b, 0, t)),
            pl.BlockSpec((1, 1, _R), lambda b, t: (b, 0, t)),
            pl.BlockSpec((1, 1, _R), lambda b, t: (b, 0, t)),
            pl.BlockSpec((1, M, 1), lambda b, t: (b, 0, 0)),
        ],
        out_specs=[
            pl.BlockSpec((1, 1), lambda b, t: (0, 0)),
            pl.BlockSpec((1, 1, _R), lambda b, t: (b, 0, t)),
            pl.BlockSpec((1, 1, _R), lambda b, t: (b, 0, t)),
        ],
        out_shape=[
            jax.ShapeDtypeStruct((1, 1), jnp.float32),
            jax.ShapeDtypeStruct((B, 1, M), jnp.float32),
            jax.ShapeDtypeStruct((B, 1, M), jnp.float32),
        ],
        scratch_shapes=[
            pltpu.VMEM((1, _R), jnp.float32),
        ],
        compiler_params=pltpu.CompilerParams(
            dimension_semantics=("arbitrary", "arbitrary"),
        ),
    )(scores_t, lt_t, clen2, cid2, cid_col)

    out2 = pl.pallas_call(
        _combine_body,
        grid=(B,),
        in_specs=[
            pl.BlockSpec((1, 1, M), lambda b: (b, 0, 0)),
            pl.BlockSpec((1, 1, M), lambda b: (b, 0, 0)),
            pl.BlockSpec((1, 1, M), lambda b: (b, 0, 0)),
        ],
        out_specs=pl.BlockSpec((1, 1), lambda b: (0, 0)),
        out_shape=jax.ShapeDtypeStruct((1, 1), jnp.float32),
        compiler_params=pltpu.CompilerParams(
            dimension_semantics=("arbitrary",),
        ),
    )(base, ediag, self_f)
    return out1[0, 0] - out2[0, 0]


# speedup vs baseline: 1.5311x; 1.5311x over previous
import jax
import jax.numpy as jnp
from jax.experimental import pallas as pl
from jax.experimental.pallas import tpu as pltpu

_B, _M, _C = 2, 4096, 16
_W = _C + _M
_R = 256
_G = 512
_NT = _M // _R


def _loss_body(scores_ref, lt_ref, clen_ref, cid_tile_ref, cid_full_ref,
               cid_col_ref, out_ref, csize_ref, sdiag_ref):
    b = pl.program_id(0)
    t = pl.program_id(1)

    @pl.when(t == 0)
    def _build():
        cid_all = cid_full_ref[0]
        gid = jax.lax.broadcasted_iota(jnp.int32, (_G, _M), 0)
        hit = gid == cid_all
        csize_ref[...] = jnp.sum(jnp.where(hit, 1.0, 0.0), axis=1, keepdims=True)

    rr = jax.lax.broadcasted_iota(jnp.int32, (_R, _R), 0)
    cc = jax.lax.broadcasted_iota(jnp.int32, (_R, _R), 1)
    eye_rr = rr == cc
    for k in range(_NT):
        @pl.when(t == k)
        def _extract(k=k):
            win = scores_ref[0, (_C + k * _R):(_C + (k + 1) * _R), :]
            sdiag_ref[...] = jnp.sum(jnp.where(eye_rr, win, 0.0), axis=0,
                                     keepdims=True)

    s = scores_ref[0]
    lt = lt_ref[0]
    clen = clen_ref[0]
    cid_row = cid_tile_ref[0]
    cid_col = cid_col_ref[0]

    m = jnp.max(s, axis=0, keepdims=True)
    e = jnp.exp(s - m)
    sum_e = jnp.sum(e, axis=0, keepdims=True)

    e_c = e[_C:, :]
    same = cid_col == cid_row
    sum_same_e = jnp.sum(jnp.where(same, e_c, 0.0), axis=0, keepdims=True)

    e_diag = jnp.exp(sdiag_ref[...] - m)
    sum_mates_e = jnp.maximum(sum_same_e - e_diag, 0.0)

    gidr = jax.lax.broadcasted_iota(jnp.int32, (_G, _R), 0)
    row_oh = gidr == cid_row
    cnt_same = jnp.sum(jnp.where(row_oh, csize_ref[...], 0.0), axis=0,
                       keepdims=True)

    c16 = jax.lax.broadcasted_iota(jnp.int32, (_C, _R), 0)
    e_l = e[:_C, :]
    link_valid = c16 < clen
    sum_inv_l = jnp.sum(jnp.where(link_valid, 0.0, e_l), axis=0, keepdims=True)
    gold_l = jnp.logical_and(lt != 0, link_valid)
    sum_gold_l = jnp.sum(jnp.where(gold_l, e_l, 0.0), axis=0, keepdims=True)
    cnt_gold_l = jnp.sum(jnp.where(gold_l, 1.0, 0.0), axis=0, keepdims=True)

    num_found = (cnt_same - 1.0) + cnt_gold_l
    self_f = jnp.where(num_found == 0.0, 1.0, 0.0)

    sum_all = sum_e - sum_inv_l
    sum_gold = sum_mates_e + self_f * e_diag + sum_gold_l

    contrib = jnp.sum(jnp.log(sum_all) - jnp.log(sum_gold), axis=1, keepdims=True)

    @pl.when(jnp.logical_and(b == 0, t == 0))
    def _init():
        out_ref[...] = jnp.zeros((1, 1), jnp.float32)

    out_ref[...] += contrib


@jax.jit
def kernel(scores, linker_targets, candidate_lengths, cluster_ids):
    B, M, W = scores.shape
    C = W - M
    scores_t = jnp.transpose(scores, (0, 2, 1))
    lt_t = jnp.transpose(linker_targets, (0, 2, 1))
    clen2 = candidate_lengths.reshape(B, 1, M)
    cid2 = cluster_ids.reshape(B, 1, M)
    cid_col = cluster_ids.reshape(B, M, 1)

    grid = (B, M // _R)
    out = pl.pallas_call(
        _loss_body,
        grid=grid,
        in_specs=[
            pl.BlockSpec((1, W, _R), lambda b, t: (b, 0, t)),
            pl.BlockSpec((1, C, _R), lambda b, t: (b, 0, t)),
            pl.BlockSpec((1, 1, _R), lambda b, t: (b, 0, t)),
            pl.BlockSpec((1, 1, _R), lambda b, t: (b, 0, t)),
            pl.BlockSpec((1, 1, M), lambda b, t: (b, 0, 0)),
            pl.BlockSpec((1, M, 1), lambda b, t: (b, 0, 0)),
        ],
        out_specs=pl.BlockSpec((1, 1), lambda b, t: (0, 0)),
        out_shape=jax.ShapeDtypeStruct((1, 1), jnp.float32),
        scratch_shapes=[
            pltpu.VMEM((_G, 1), jnp.float32),
            pltpu.VMEM((1, _R), jnp.float32),
        ],
        compiler_params=pltpu.CompilerParams(
            dimension_semantics=("arbitrary", "arbitrary"),
        ),
    )(scores_t, lt_t, clen2, cid2, cid2, cid_col)
    return out[0, 0]
